# Initial kernel scaffold; baseline (speedup 1.0000x reference)
import functools

import jax
import jax.numpy as jnp
from jax import lax
from jax.experimental import pallas as pl
from jax.experimental.pallas import tpu as pltpu
from jax.experimental.pallas import tpu_sc as plsc

F = 32
L = 16
CB = 16


@functools.cache
def _build(B, H):
    info = plsc.get_sparse_core_info()
    NC, NS = info.num_cores, info.num_subcores
    NW = NC * NS
    assert B % (NW * CB) == 0
    BPW = B // NW
    n_chunks = BPW // CB

    mesh = plsc.VectorSubcoreMesh(core_axis_name="c", subcore_axis_name="s")

    @functools.partial(
        pl.kernel,
        mesh=mesh,
        out_type=jax.ShapeDtypeStruct((B,), jnp.float32),
        scratch_types=[
            pltpu.VMEM((CB,), jnp.int32),
            pltpu.VMEM((CB,), jnp.int32),
            pltpu.VMEM((CB, H), jnp.int32),
            pltpu.VMEM((CB, F), jnp.float32),
            pltpu.VMEM((CB, F), jnp.float32),
            pltpu.VMEM((CB, H, F), jnp.float32),
            pltpu.VMEM((B // 32,), jnp.float32),
            pltpu.SemaphoreType.DMA,
        ],
    )
    def _k(th_h, be_h, rh_h, us_h, it_h, cx_h, out_h,
           uidx, iidx, cidx, th_v, be_v, rho_v, outb, sem):
        wid = lax.axis_index("s") * NC + lax.axis_index("c")
        base = pl.multiple_of(wid * BPW, BPW)

        def chunk_body(c, carry):
            gb = pl.multiple_of(base + c * CB, CB)
            pltpu.sync_copy(us_h.at[pl.ds(gb, CB)], uidx)
            pltpu.sync_copy(it_h.at[pl.ds(gb, CB)], iidx)
            pltpu.sync_copy(cx_h.at[pl.ds(gb, CB)], cidx)
            cp1 = pltpu.async_copy(th_h.at[uidx], th_v, sem)
            cp2 = pltpu.async_copy(be_h.at[iidx], be_v, sem)
            cp3 = pltpu.async_copy(rh_h.at[cidx], rho_v, sem)
            cp1.wait()
            cp2.wait()
            cp3.wait()

            def e_body(e, carry2):
                acc0 = th_v[e, pl.ds(0, L)]
                acc1 = th_v[e, pl.ds(L, L)]
                for h in range(H):
                    acc0 = acc0 + rho_v[e, h, pl.ds(0, L)]
                    acc1 = acc1 + rho_v[e, h, pl.ds(L, L)]
                p = be_v[e, pl.ds(0, L)] * acc0 + be_v[e, pl.ds(L, L)] * acc1
                outb[c * CB + e] = jnp.sum(p)
                return carry2

            lax.fori_loop(0, CB, e_body, 0)
            return carry

        lax.fori_loop(0, n_chunks, chunk_body, 0)

        def sig_body(i, carry):
            off = pl.multiple_of(i * L, L)
            x = outb[pl.ds(off, L)]
            outb[pl.ds(off, L)] = 1.0 / (1.0 + jnp.exp(-x))
            return carry

        lax.fori_loop(0, BPW // L, sig_body, 0)
        pltpu.sync_copy(outb, out_h.at[pl.ds(base, BPW)])

    return _k


def kernel(theta, beta, rho, user, item, contexts):
    B, H = contexts.shape
    return _build(B, H)(theta, beta, rho, user, item, contexts)


# speedup vs baseline: 1.2070x; 1.2070x over previous
import functools

import jax
import jax.numpy as jnp
from jax import lax
from jax.experimental import pallas as pl
from jax.experimental.pallas import tpu as pltpu
from jax.experimental.pallas import tpu_sc as plsc

F = 32
L = 16
CB = 16
GR = 80


@functools.cache
def _build(B, H):
    info = plsc.get_sparse_core_info()
    NC, NS = info.num_cores, info.num_subcores
    NW = NC * NS
    assert B % (NW * CB) == 0
    BPW = B // NW
    n_chunks = BPW // CB

    mesh = plsc.VectorSubcoreMesh(core_axis_name="c", subcore_axis_name="s")

    @functools.partial(
        pl.kernel,
        mesh=mesh,
        compiler_params=pltpu.CompilerParams(
            needs_layout_passes=False, use_tc_tiling_on_sc=False),
        out_type=jax.ShapeDtypeStruct((B,), jnp.float32),
        scratch_types=[
            pltpu.VMEM((CB,), jnp.int32),
            pltpu.VMEM((CB,), jnp.int32),
            pltpu.VMEM((CB * H,), jnp.int32),
            pltpu.VMEM((CB, F), jnp.float32),
            pltpu.VMEM((CB, F), jnp.float32),
            pltpu.VMEM((CB * H, F), jnp.float32),
            pltpu.VMEM((CB, L), jnp.float32),
            pltpu.VMEM((B // 32,), jnp.float32),
            pltpu.SemaphoreType.DMA,
        ],
    )
    def _k(th_h, be_h, rh_h, us_h, it_h, cx_h, out_h,
           uidx, iidx, cidx, th_v, be_v, rho_v, q_v, outb, sem):
        wid = lax.axis_index("s") * NC + lax.axis_index("c")
        base = pl.multiple_of(wid * BPW, BPW)

        def chunk_body(c, carry):
            gb = pl.multiple_of(base + c * CB, CB)
            pltpu.sync_copy(us_h.at[pl.ds(gb, CB)], uidx)
            pltpu.sync_copy(it_h.at[pl.ds(gb, CB)], iidx)
            gbc = pl.multiple_of((base + c * CB) * H, CB * H)
            pltpu.sync_copy(cx_h.at[pl.ds(gbc, CB * H)], cidx)
            cps = [
                pltpu.async_copy(th_h.at[uidx], th_v, sem),
                pltpu.async_copy(be_h.at[iidx], be_v, sem),
            ]
            for g in range(0, CB * H, GR):
                cps.append(pltpu.async_copy(
                    rh_h.at[cidx.at[pl.ds(g, GR)]],
                    rho_v.at[pl.ds(g, GR)], sem))
            for cp in cps:
                cp.wait()

            def e_body(e, carry2):
                acc0 = th_v[e, pl.ds(0, L)]
                acc1 = th_v[e, pl.ds(L, L)]
                for h in range(H):
                    acc0 = acc0 + rho_v[e * H + h, pl.ds(0, L)]
                    acc1 = acc1 + rho_v[e * H + h, pl.ds(L, L)]
                q_v[e, pl.ds(0, L)] = (be_v[e, pl.ds(0, L)] * acc0
                                       + be_v[e, pl.ds(L, L)] * acc1)
                return carry2

            lax.fori_loop(0, CB, e_body, 0)

            lanes = lax.iota(jnp.int32, L)
            svec = jnp.zeros((L,), jnp.float32)
            for j in range(L):
                svec = svec + plsc.load_gather(
                    q_v, [lanes, jnp.full((L,), j, jnp.int32)])
            outb[pl.ds(pl.multiple_of(c * CB, CB), CB)] = svec
            return carry

        lax.fori_loop(0, n_chunks, chunk_body, 0)

        def sig_body(i, carry):
            off = pl.multiple_of(i * L, L)
            x = outb[pl.ds(off, L)]
            outb[pl.ds(off, L)] = 1.0 / (1.0 + jnp.exp(-x))
            return carry

        lax.fori_loop(0, BPW // L, sig_body, 0)
        pltpu.sync_copy(outb, out_h.at[pl.ds(base, BPW)])

    return _k


def kernel(theta, beta, rho, user, item, contexts):
    B, H = contexts.shape
    return _build(B, H)(theta, beta, rho, user, item,
                        contexts.reshape(B * H))
